# Initial kernel scaffold; baseline (speedup 1.0000x reference)
import jax
import jax.numpy as jnp
from jax.experimental import pallas as pl


def kernel(pb, idx):
    raise NotImplementedError("write your pallas kernel here")


# speedup vs baseline: 2.7208x; 2.7208x over previous
import functools

import jax
import jax.numpy as jnp
from jax import lax
from jax.experimental import pallas as pl
from jax.experimental.pallas import tpu as pltpu
from jax.experimental.pallas import tpu_sc as plsc

N = 257
C = N * N
HEAD = 16
TABLE_WORDS = 964 * HEAD
NC, NS = 2, 16
NW = NC * NS
CHUNK = 2176
C_PAD = NW * CHUNK


def _bias_gather(idx_hbm, pb_hbm, out_hbm, idx_v, pb_v, out_v):
    wid = lax.axis_index("s") * NC + lax.axis_index("c")
    base = wid * CHUNK
    pltpu.sync_copy(idx_hbm.at[pl.ds(base, CHUNK)], idx_v)
    pltpu.sync_copy(pb_hbm, pb_v)

    def body(j, carry):
        off = j * 16
        ids = idx_v[pl.ds(off, 16)] * HEAD
        for h in range(HEAD):
            out_v[h, pl.ds(off, 16)] = plsc.load_gather(pb_v, [ids + h])
        return carry

    lax.fori_loop(0, CHUNK // 16, body, 0)
    pltpu.sync_copy(out_v, out_hbm.at[:, pl.ds(base, CHUNK)])


@functools.partial(
    pl.kernel,
    out_type=jax.ShapeDtypeStruct((HEAD, C_PAD), jnp.float32),
    mesh=plsc.VectorSubcoreMesh(
        core_axis_name="c", subcore_axis_name="s", num_cores=NC, num_subcores=NS
    ),
    scratch_types=[
        pltpu.VMEM((CHUNK,), jnp.int32),
        pltpu.VMEM((TABLE_WORDS,), jnp.float32),
        pltpu.VMEM((HEAD, CHUNK), jnp.float32),
    ],
    compiler_params=pltpu.CompilerParams(needs_layout_passes=False),
)
def _bias_gather_call(idx_hbm, pb_hbm, out_hbm, idx_v, pb_v, out_v):
    _bias_gather(idx_hbm, pb_hbm, out_hbm, idx_v, pb_v, out_v)


def kernel(pb, idx):
    idx_pad = jnp.pad(jnp.ravel(idx), (0, C_PAD - C))
    out = _bias_gather_call(idx_pad, jnp.ravel(pb))
    return out[:, :C].reshape(HEAD, N, N)


# speedup vs baseline: 3.8390x; 1.4110x over previous
import functools

import jax
import jax.numpy as jnp
from jax import lax
from jax.experimental import pallas as pl
from jax.experimental.pallas import tpu as pltpu
from jax.experimental.pallas import tpu_sc as plsc

N = 257
C = N * N
HEAD = 16
TABLE = 964
TABLE_WORDS = TABLE * HEAD
NC, NS = 2, 16
NW = NC * NS
CHUNK = 2176
C_PAD = NW * CHUNK


def _bias_gather(idx_hbm, pbt_hbm, out_hbm, idx_v, pbt_v, out_v, sem_i, sem_t):
    wid = lax.axis_index("s") * NC + lax.axis_index("c")
    base = wid * CHUNK
    cp_i = pltpu.async_copy(idx_hbm.at[pl.ds(base, CHUNK)], idx_v, sem_i)
    cp_t = pltpu.async_copy(pbt_hbm, pbt_v, sem_t)
    cp_i.wait()
    cp_t.wait()

    @plsc.parallel_loop(0, CHUNK, step=16, unroll=4)
    def _(off):
        ids = idx_v[pl.ds(off, 16)]
        for h in range(HEAD):
            out_v[h, pl.ds(off, 16)] = plsc.load_gather(pbt_v, [ids + h * TABLE])

    pltpu.sync_copy(out_v, out_hbm.at[:, pl.ds(base, CHUNK)])


@functools.partial(
    pl.kernel,
    out_type=jax.ShapeDtypeStruct((HEAD, C_PAD), jnp.float32),
    mesh=plsc.VectorSubcoreMesh(
        core_axis_name="c", subcore_axis_name="s", num_cores=NC, num_subcores=NS
    ),
    scratch_types=[
        pltpu.VMEM((CHUNK,), jnp.int32),
        pltpu.VMEM((TABLE_WORDS,), jnp.float32),
        pltpu.VMEM((HEAD, CHUNK), jnp.float32),
        pltpu.SemaphoreType.DMA,
        pltpu.SemaphoreType.DMA,
    ],
    compiler_params=pltpu.CompilerParams(needs_layout_passes=False),
)
def _bias_gather_call(idx_hbm, pbt_hbm, out_hbm, idx_v, pbt_v, out_v, sem_i, sem_t):
    _bias_gather(idx_hbm, pbt_hbm, out_hbm, idx_v, pbt_v, out_v, sem_i, sem_t)


def kernel(pb, idx):
    idx_pad = jnp.pad(jnp.ravel(idx), (0, C_PAD - C))
    pbt = jnp.ravel(jnp.transpose(pb))
    out = _bias_gather_call(idx_pad, pbt)
    return out[:, :C].reshape(HEAD, N, N)
